# Initial kernel scaffold; baseline (speedup 1.0000x reference)
import jax
import jax.numpy as jnp
from jax.experimental import pallas as pl


def kernel(hidden_states, gate_w, gate_proj, up_proj, down_proj):
    raise NotImplementedError("write your pallas kernel here")


# speedup vs baseline: 1.8773x; 1.8773x over previous
import functools

import jax
import jax.numpy as jnp
from jax import lax
from jax.experimental import pallas as pl
from jax.experimental.pallas import tpu as pltpu
from jax.experimental.pallas import tpu_sc as plsc

T = 2048
D = 2048
H = 1408
E = 8
B = 256
NPAIR = 2 * T
NBUF = NPAIR + E * B
NB = NBUF // B
NW = 32
TPW = T // NW
RND = TPW // 16


def _cumsum_excl(x):
    s = x
    shift = 1
    while shift < x.shape[0]:
        pad = jnp.zeros((shift, x.shape[1]), x.dtype)
        s = s + jnp.concatenate([pad, s[:-shift, :]], axis=0)
        shift *= 2
    return s - x


def _route_body(x_ref, gw_ref, pos0_ref, pos1_ref, w1_ref, be_ref, act_ref):
    x = x_ref[...]
    gw = gw_ref[...]
    logits = lax.dot_general(x, gw, (((1,), (1,)), ((), ())),
                             preferred_element_type=jnp.float32)
    iota8 = lax.broadcasted_iota(jnp.int32, (T, E), 1)
    m1 = jnp.max(logits, axis=1, keepdims=True)
    e1 = jnp.min(jnp.where(logits >= m1, iota8, E), axis=1, keepdims=True)
    oh1 = iota8 == e1
    l2 = jnp.where(oh1, -jnp.inf, logits)
    m2 = jnp.max(l2, axis=1, keepdims=True)
    e2 = jnp.min(jnp.where(l2 >= m2, iota8, E), axis=1, keepdims=True)
    oh2 = iota8 == e2
    w1_ref[...] = jnp.broadcast_to(1.0 / (1.0 + jnp.exp(m2 - m1)), (T, 16))

    o1 = oh1.astype(jnp.int32)
    o2 = oh2.astype(jnp.int32)
    c1 = _cumsum_excl(o1)
    tot1 = jnp.sum(o1, axis=0, keepdims=True)
    c2 = _cumsum_excl(o2)
    tot2 = jnp.sum(o2, axis=0, keepdims=True)
    counts = tot1 + tot2

    lane8 = lax.broadcasted_iota(jnp.int32, (1, E), 1)
    bstart = lax.broadcasted_iota(jnp.int32, (1, NB), 1) * B
    off_vec = jnp.zeros((1, E), jnp.int32)
    be = jnp.zeros((1, NB), jnp.int32)
    off_s = jnp.int32(0)
    for e in range(E):
        off_vec = off_vec + jnp.where(lane8 == e, off_s, 0)
        if e > 0:
            be = be + (bstart >= off_s).astype(jnp.int32)
        c_e = jnp.sum(jnp.where(lane8 == e, counts, 0))
        off_s = ((off_s + c_e + B - 1) // B) * B
    act = (bstart < off_s).astype(jnp.int32)

    pos0_ref[...] = jnp.sum(o1 * (off_vec + c1), axis=1, keepdims=True)
    pos1_ref[...] = jnp.sum(o2 * (off_vec + tot1 + c2), axis=1, keepdims=True)
    be_ref[...] = be
    act_ref[...] = act


def _route(x, gate_w):
    return pl.pallas_call(
        _route_body,
        out_shape=(
            jax.ShapeDtypeStruct((T, 1), jnp.int32),
            jax.ShapeDtypeStruct((T, 1), jnp.int32),
            jax.ShapeDtypeStruct((T, 16), jnp.float32),
            jax.ShapeDtypeStruct((1, NB), jnp.int32),
            jax.ShapeDtypeStruct((1, NB), jnp.int32),
        ),
        compiler_params=pltpu.CompilerParams(
            vmem_limit_bytes=100 * 1024 * 1024),
    )(x, gate_w)


def _dispatch_body(x_hbm, posidx_hbm, xg_hbm, idx_v, buf0, buf1, sem0, sem1):
    wid = lax.axis_index("s") * 2 + lax.axis_index("c")
    base = wid * TPW
    pltpu.sync_copy(posidx_hbm.at[wid], idx_v)
    cps = []
    for r in range(RND):
        buf = buf0 if r % 2 == 0 else buf1
        if len(cps) >= 4:
            cps.pop(0).wait()
            cps.pop(0).wait()
        pltpu.sync_copy(x_hbm.at[pl.ds(base + r * 16, 16), :], buf)
        cps.append(pltpu.async_copy(buf, xg_hbm.at[idx_v.at[r]], sem0))
        cps.append(pltpu.async_copy(buf, xg_hbm.at[idx_v.at[RND + r]], sem1))
    for cp in cps:
        cp.wait()


def _dispatch(x, posidx):
    mesh = plsc.VectorSubcoreMesh(core_axis_name="c", subcore_axis_name="s")
    fn = functools.partial(
        pl.kernel,
        mesh=mesh,
        out_type=jax.ShapeDtypeStruct((NBUF, D), jnp.float32),
        scratch_types=[
            pltpu.VMEM((2 * RND, 16), jnp.int32),
            pltpu.VMEM((16, D), jnp.float32),
            pltpu.VMEM((16, D), jnp.float32),
            pltpu.SemaphoreType.DMA,
            pltpu.SemaphoreType.DMA,
        ],
    )(_dispatch_body)
    return fn(x, posidx)


def _mlp1_body(be_s, act_s, xg_ref, g_ref, u_ref, h_ref):
    b = pl.program_id(0)

    @pl.when(act_s[0, b] > 0)
    def _():
        xb = xg_ref[...].astype(jnp.bfloat16)
        gw = g_ref[0].astype(jnp.bfloat16)
        uw = u_ref[0].astype(jnp.bfloat16)
        g = lax.dot_general(xb, gw, (((1,), (1,)), ((), ())),
                            preferred_element_type=jnp.float32)
        u = lax.dot_general(xb, uw, (((1,), (1,)), ((), ())),
                            preferred_element_type=jnp.float32)
        h_ref[...] = (g * (1.0 / (1.0 + jnp.exp(-g))) * u).astype(jnp.bfloat16)


def _mlp2_body(be_s, act_s, h_ref, d_ref, out_ref):
    b = pl.program_id(0)

    @pl.when(act_s[0, b] > 0)
    def _():
        dw = d_ref[0].astype(jnp.bfloat16)
        out_ref[...] = lax.dot_general(h_ref[...], dw, (((1,), (1,)), ((), ())),
                                       preferred_element_type=jnp.float32)


def _mlp(be, act, xg, gate_proj, up_proj, down_proj):
    h = pl.pallas_call(
        _mlp1_body,
        grid_spec=pltpu.PrefetchScalarGridSpec(
            num_scalar_prefetch=2,
            grid=(NB,),
            in_specs=[
                pl.BlockSpec((B, D), lambda b, be, act: (b, 0)),
                pl.BlockSpec((1, H, D), lambda b, be, act: (be[0, b], 0, 0)),
                pl.BlockSpec((1, H, D), lambda b, be, act: (be[0, b], 0, 0)),
            ],
            out_specs=pl.BlockSpec((B, H), lambda b, be, act: (b, 0)),
        ),
        out_shape=jax.ShapeDtypeStruct((NBUF, H), jnp.bfloat16),
        compiler_params=pltpu.CompilerParams(
            dimension_semantics=("arbitrary",)),
    )(be, act, xg, gate_proj, up_proj)
    return pl.pallas_call(
        _mlp2_body,
        grid_spec=pltpu.PrefetchScalarGridSpec(
            num_scalar_prefetch=2,
            grid=(NB,),
            in_specs=[
                pl.BlockSpec((B, H), lambda b, be, act: (b, 0)),
                pl.BlockSpec((1, D, H), lambda b, be, act: (be[0, b], 0, 0)),
            ],
            out_specs=pl.BlockSpec((B, D), lambda b, be, act: (b, 0)),
        ),
        out_shape=jax.ShapeDtypeStruct((NBUF, D), jnp.float32),
        compiler_params=pltpu.CompilerParams(
            dimension_semantics=("arbitrary",)),
    )(be, act, h, down_proj)


def _combine_body(po_hbm, posidx_hbm, wr_hbm, out_hbm,
                  idx_v, w_v, buf0, buf1, obuf, sem0, sem1):
    wid = lax.axis_index("s") * 2 + lax.axis_index("c")
    base = wid * TPW
    pltpu.sync_copy(posidx_hbm.at[wid], idx_v)
    for r in range(RND):
        cp0 = pltpu.async_copy(po_hbm.at[idx_v.at[r]], buf0, sem0)
        cp1 = pltpu.async_copy(po_hbm.at[idx_v.at[RND + r]], buf1, sem1)
        pltpu.sync_copy(wr_hbm.at[pl.ds(base + r * 16, 16), :], w_v)
        cp0.wait()
        cp1.wait()
        for i in range(16):
            wv = w_v[i, :]
            wv2 = 1.0 - wv

            def body(c, _):
                sl = pl.ds(c * 16, 16)
                obuf[i, sl] = buf0[i, sl] * wv + buf1[i, sl] * wv2
                return 0

            lax.fori_loop(0, D // 16, body, 0)
        pltpu.sync_copy(obuf, out_hbm.at[pl.ds(base + r * 16, 16), :])


def _combine(po, posidx, wr):
    mesh = plsc.VectorSubcoreMesh(core_axis_name="c", subcore_axis_name="s")
    fn = functools.partial(
        pl.kernel,
        mesh=mesh,
        out_type=jax.ShapeDtypeStruct((T, D), jnp.float32),
        scratch_types=[
            pltpu.VMEM((2 * RND, 16), jnp.int32),
            pltpu.VMEM((16, 16), jnp.float32),
            pltpu.VMEM((16, D), jnp.float32),
            pltpu.VMEM((16, D), jnp.float32),
            pltpu.VMEM((16, D), jnp.float32),
            pltpu.SemaphoreType.DMA,
            pltpu.SemaphoreType.DMA,
        ],
    )(_combine_body)
    return fn(po, posidx, wr)


def kernel(hidden_states, gate_w, gate_proj, up_proj, down_proj):
    bsz, seq, dim = hidden_states.shape
    x = hidden_states.reshape(T, D)
    pos0, pos1, w1, be, act = _route(x, gate_w)
    posidx = jnp.concatenate(
        [pos0.reshape(NW, RND, 16), pos1.reshape(NW, RND, 16)], axis=1)
    xg = _dispatch(x, posidx)
    po = _mlp(be, act, xg, gate_proj, up_proj, down_proj)
    out = _combine(po, posidx, w1)
    return out.reshape(bsz, seq, dim)


# speedup vs baseline: 1.9729x; 1.0509x over previous
import functools

import jax
import jax.numpy as jnp
from jax import lax
from jax.experimental import pallas as pl
from jax.experimental.pallas import tpu as pltpu
from jax.experimental.pallas import tpu_sc as plsc

T = 2048
D = 2048
H = 1408
E = 8
B = 256
NPAIR = 2 * T
NBUF = NPAIR + E * B
NB = NBUF // B
NW = 32
TPW = T // NW
RSZ = 8
RND = TPW // RSZ


def _cumsum_excl(x):
    s = x
    shift = 1
    while shift < x.shape[0]:
        pad = jnp.zeros((shift, x.shape[1]), x.dtype)
        s = s + jnp.concatenate([pad, s[:-shift, :]], axis=0)
        shift *= 2
    return s - x


def _route_body(x_ref, gw_ref, pos0_ref, pos1_ref, w1_ref, be_ref, act_ref):
    x = x_ref[...]
    gw = gw_ref[...]
    logits = lax.dot_general(x, gw, (((1,), (1,)), ((), ())),
                             preferred_element_type=jnp.float32)
    iota8 = lax.broadcasted_iota(jnp.int32, (T, E), 1)
    m1 = jnp.max(logits, axis=1, keepdims=True)
    e1 = jnp.min(jnp.where(logits >= m1, iota8, E), axis=1, keepdims=True)
    oh1 = iota8 == e1
    l2 = jnp.where(oh1, -jnp.inf, logits)
    m2 = jnp.max(l2, axis=1, keepdims=True)
    e2 = jnp.min(jnp.where(l2 >= m2, iota8, E), axis=1, keepdims=True)
    oh2 = iota8 == e2
    w1_ref[...] = jnp.broadcast_to(1.0 / (1.0 + jnp.exp(m2 - m1)), (T, 16))

    o1 = oh1.astype(jnp.int32)
    o2 = oh2.astype(jnp.int32)
    c1 = _cumsum_excl(o1)
    tot1 = jnp.sum(o1, axis=0, keepdims=True)
    c2 = _cumsum_excl(o2)
    tot2 = jnp.sum(o2, axis=0, keepdims=True)
    counts = tot1 + tot2

    lane8 = lax.broadcasted_iota(jnp.int32, (1, E), 1)
    bstart = lax.broadcasted_iota(jnp.int32, (1, NB), 1) * B
    off_vec = jnp.zeros((1, E), jnp.int32)
    be = jnp.zeros((1, NB), jnp.int32)
    off_s = jnp.int32(0)
    for e in range(E):
        off_vec = off_vec + jnp.where(lane8 == e, off_s, 0)
        if e > 0:
            be = be + (bstart >= off_s).astype(jnp.int32)
        c_e = jnp.sum(jnp.where(lane8 == e, counts, 0))
        off_s = ((off_s + c_e + B - 1) // B) * B
    act = (bstart < off_s).astype(jnp.int32)

    pos0_ref[...] = jnp.sum(o1 * (off_vec + c1), axis=1, keepdims=True)
    pos1_ref[...] = jnp.sum(o2 * (off_vec + tot1 + c2), axis=1, keepdims=True)
    be_ref[...] = be
    act_ref[...] = act


def _route(x, gate_w):
    return pl.pallas_call(
        _route_body,
        out_shape=(
            jax.ShapeDtypeStruct((T, 1), jnp.int32),
            jax.ShapeDtypeStruct((T, 1), jnp.int32),
            jax.ShapeDtypeStruct((T, 16), jnp.float32),
            jax.ShapeDtypeStruct((1, NB), jnp.int32),
            jax.ShapeDtypeStruct((1, NB), jnp.int32),
        ),
        compiler_params=pltpu.CompilerParams(
            vmem_limit_bytes=100 * 1024 * 1024),
    )(x, gate_w)


def _dispatch_body(x_hbm, posidx_hbm, xg_hbm, idx_v, buf0, buf1,
                   s00, s01, s10, s11):
    wid = lax.axis_index("s") * 2 + lax.axis_index("c")
    base = wid * TPW
    bufs = (buf0, buf1)
    sems = ((s00, s01), (s10, s11))
    pltpu.sync_copy(posidx_hbm.at[wid], idx_v)
    cps = [None, None]
    for r in range(RND):
        slot = r % 2
        if cps[slot] is not None:
            cps[slot][0].wait()
            cps[slot][1].wait()
        pltpu.sync_copy(x_hbm.at[pl.ds(base + r * RSZ, RSZ), :], bufs[slot])
        cps[slot] = (
            pltpu.async_copy(bufs[slot], xg_hbm.at[idx_v.at[r]],
                             sems[slot][0]),
            pltpu.async_copy(bufs[slot], xg_hbm.at[idx_v.at[RND + r]],
                             sems[slot][1]),
        )
    for cp in cps:
        cp[0].wait()
        cp[1].wait()


def _dispatch(x, posidx):
    mesh = plsc.VectorSubcoreMesh(core_axis_name="c", subcore_axis_name="s")
    fn = functools.partial(
        pl.kernel,
        mesh=mesh,
        out_type=jax.ShapeDtypeStruct((NBUF, D), jnp.float32),
        scratch_types=[
            pltpu.VMEM((2 * RND, RSZ), jnp.int32),
            pltpu.VMEM((RSZ, D), jnp.float32),
            pltpu.VMEM((RSZ, D), jnp.float32),
            pltpu.SemaphoreType.DMA,
            pltpu.SemaphoreType.DMA,
            pltpu.SemaphoreType.DMA,
            pltpu.SemaphoreType.DMA,
        ],
    )(_dispatch_body)
    return fn(x, posidx)


def _mlp1_body(be_s, act_s, xg_ref, g_ref, u_ref, h_ref):
    b = pl.program_id(0)

    @pl.when(act_s[0, b] > 0)
    def _():
        xb = xg_ref[...].astype(jnp.bfloat16)
        gw = g_ref[0].astype(jnp.bfloat16)
        uw = u_ref[0].astype(jnp.bfloat16)
        g = lax.dot_general(xb, gw, (((1,), (1,)), ((), ())),
                            preferred_element_type=jnp.float32)
        u = lax.dot_general(xb, uw, (((1,), (1,)), ((), ())),
                            preferred_element_type=jnp.float32)
        h_ref[...] = (g * (1.0 / (1.0 + jnp.exp(-g))) * u).astype(jnp.bfloat16)


def _mlp2_body(be_s, act_s, h_ref, d_ref, out_ref, dwb_ref):
    b = pl.program_id(0)

    @pl.when(act_s[0, b] > 0)
    def _():
        prev = be_s[0, jnp.maximum(b - 1, 0)]
        @pl.when(jnp.logical_or(b == 0, be_s[0, b] != prev))
        def _():
            dwb_ref[...] = d_ref[0].astype(jnp.bfloat16)
        out_ref[...] = lax.dot_general(h_ref[...], dwb_ref[...],
                                       (((1,), (1,)), ((), ())),
                                       preferred_element_type=jnp.float32)


def _mlp(be, act, xg, gate_proj, up_proj, down_proj):
    h = pl.pallas_call(
        _mlp1_body,
        grid_spec=pltpu.PrefetchScalarGridSpec(
            num_scalar_prefetch=2,
            grid=(NB,),
            in_specs=[
                pl.BlockSpec((B, D), lambda b, be, act: (b, 0)),
                pl.BlockSpec((1, H, D), lambda b, be, act: (be[0, b], 0, 0)),
                pl.BlockSpec((1, H, D), lambda b, be, act: (be[0, b], 0, 0)),
            ],
            out_specs=pl.BlockSpec((B, H), lambda b, be, act: (b, 0)),
        ),
        out_shape=jax.ShapeDtypeStruct((NBUF, H), jnp.bfloat16),
        compiler_params=pltpu.CompilerParams(
            dimension_semantics=("arbitrary",)),
    )(be, act, xg, gate_proj, up_proj)
    return pl.pallas_call(
        _mlp2_body,
        grid_spec=pltpu.PrefetchScalarGridSpec(
            num_scalar_prefetch=2,
            grid=(NB,),
            in_specs=[
                pl.BlockSpec((B, H), lambda b, be, act: (b, 0)),
                pl.BlockSpec((1, D, H), lambda b, be, act: (be[0, b], 0, 0)),
            ],
            out_specs=pl.BlockSpec((B, D), lambda b, be, act: (b, 0)),
            scratch_shapes=[pltpu.VMEM((D, H), jnp.bfloat16)],
        ),
        out_shape=jax.ShapeDtypeStruct((NBUF, D), jnp.float32),
        compiler_params=pltpu.CompilerParams(
            dimension_semantics=("arbitrary",)),
    )(be, act, h, down_proj)


def _combine_body(po_hbm, posidx_hbm, wr_hbm, out_hbm,
                  idx_v, w_v, g0a, g0b, g1a, g1b, oba, obb,
                  sga, sgb, swa, swb):
    wid = lax.axis_index("s") * 2 + lax.axis_index("c")
    base = wid * TPW
    g0 = (g0a, g0b)
    g1 = (g1a, g1b)
    ob = (oba, obb)
    sg = (sga, sgb)
    sw = (swa, swb)
    pltpu.sync_copy(posidx_hbm.at[wid], idx_v)

    def start(r, slot):
        return (pltpu.async_copy(po_hbm.at[idx_v.at[r]], g0[slot], sg[slot]),
                pltpu.async_copy(po_hbm.at[idx_v.at[RND + r]], g1[slot],
                                 sg[slot]))

    pending = start(0, 0)
    owr = [None, None]
    for r in range(RND):
        slot = r % 2
        nxt = start(r + 1, 1 - slot) if r + 1 < RND else None
        pltpu.sync_copy(wr_hbm.at[pl.ds(base + r * RSZ, RSZ), :], w_v)
        pending[0].wait()
        pending[1].wait()
        if owr[slot] is not None:
            owr[slot].wait()
        wvs = [w_v[i, :] for i in range(RSZ)]
        wv2 = [1.0 - w for w in wvs]
        a = g0[slot]
        bb = g1[slot]
        o = ob[slot]

        def body(c, _):
            sl = pl.ds(c * 16, 16)
            for i in range(RSZ):
                o[i, sl] = a[i, sl] * wvs[i] + bb[i, sl] * wv2[i]
            return 0

        lax.fori_loop(0, D // 16, body, 0)
        owr[slot] = pltpu.async_copy(
            o, out_hbm.at[pl.ds(base + r * RSZ, RSZ), :], sw[slot])
        pending = nxt
    for cp in owr:
        if cp is not None:
            cp.wait()


def _combine(po, posidx, wr):
    mesh = plsc.VectorSubcoreMesh(core_axis_name="c", subcore_axis_name="s")
    fn = functools.partial(
        pl.kernel,
        mesh=mesh,
        out_type=jax.ShapeDtypeStruct((T, D), jnp.float32),
        scratch_types=[
            pltpu.VMEM((2 * RND, RSZ), jnp.int32),
            pltpu.VMEM((RSZ, 16), jnp.float32),
            pltpu.VMEM((RSZ, D), jnp.float32),
            pltpu.VMEM((RSZ, D), jnp.float32),
            pltpu.VMEM((RSZ, D), jnp.float32),
            pltpu.VMEM((RSZ, D), jnp.float32),
            pltpu.VMEM((RSZ, D), jnp.float32),
            pltpu.VMEM((RSZ, D), jnp.float32),
            pltpu.SemaphoreType.DMA,
            pltpu.SemaphoreType.DMA,
            pltpu.SemaphoreType.DMA,
            pltpu.SemaphoreType.DMA,
        ],
    )(_combine_body)
    return fn(po, posidx, wr)


def kernel(hidden_states, gate_w, gate_proj, up_proj, down_proj):
    bsz, seq, dim = hidden_states.shape
    x = hidden_states.reshape(T, D)
    pos0, pos1, w1, be, act = _route(x, gate_w)
    posidx = jnp.concatenate(
        [pos0.reshape(NW, RND, RSZ), pos1.reshape(NW, RND, RSZ)], axis=1)
    xg = _dispatch(x, posidx)
    po = _mlp(be, act, xg, gate_proj, up_proj, down_proj)
    out = _combine(po, posidx, w1)
    return out.reshape(bsz, seq, dim)
